# Initial kernel scaffold; baseline (speedup 1.0000x reference)
import jax
import jax.numpy as jnp
from jax.experimental import pallas as pl


def kernel(token_ids, weight):
    raise NotImplementedError("write your pallas kernel here")


# speedup vs baseline: 1.6863x; 1.6863x over previous
import functools

import jax
import jax.numpy as jnp
from jax import lax
from jax.experimental import pallas as pl
from jax.experimental.pallas import tpu as pltpu
from jax.experimental.pallas import tpu_sc as plsc

_NUM_CORES = 2
_NUM_SUBCORES = 16
_NW = _NUM_CORES * _NUM_SUBCORES
_CHUNK = 128


@functools.lru_cache(maxsize=None)
def _make_gather(b_total: int, d: int):
    assert b_total % (_NW * _CHUNK) == 0
    b_per_w = b_total // _NW
    n_chunks = b_per_w // _CHUNK
    mesh = plsc.VectorSubcoreMesh(core_axis_name="c", subcore_axis_name="s")

    @functools.partial(
        pl.kernel,
        out_type=jax.ShapeDtypeStruct((b_total, d), jnp.float32),
        mesh=mesh,
        scratch_types=[
            pltpu.VMEM((n_chunks, _CHUNK), jnp.int32),
            pltpu.VMEM((_CHUNK, d), jnp.float32),
            pltpu.SemaphoreType.DMA,
        ],
        compiler_params=pltpu.CompilerParams(use_tc_tiling_on_sc=False),
    )
    def gather_kernel(idx_hbm, table_hbm, out_hbm, idx_v, rows_v, gsem):
        wid = lax.axis_index("s") * _NUM_CORES + lax.axis_index("c")
        base = wid * b_per_w
        pltpu.sync_copy(idx_hbm.at[wid], idx_v)

        def body(i, carry):
            pltpu.async_copy(table_hbm.at[idx_v.at[i]], rows_v, gsem).wait()
            pltpu.sync_copy(rows_v, out_hbm.at[pl.ds(base + i * _CHUNK, _CHUNK)])
            return carry

        lax.fori_loop(0, n_chunks, body, 0)

    return gather_kernel


def kernel(token_ids, weight):
    b, t = token_ids.shape
    d = weight.shape[1]
    idx = token_ids.astype(jnp.int32).reshape(_NW, -1, _CHUNK)
    out = _make_gather(b * t, d)(idx, weight)
    return out.reshape(b, t, d)


# speedup vs baseline: 1.8628x; 1.1046x over previous
import functools

import jax
import jax.numpy as jnp
from jax import lax
from jax.experimental import pallas as pl
from jax.experimental.pallas import tpu as pltpu
from jax.experimental.pallas import tpu_sc as plsc

_NUM_CORES = 2
_NUM_SUBCORES = 16
_NW = _NUM_CORES * _NUM_SUBCORES
_CHUNK = 128
_NBUF = 4


@functools.lru_cache(maxsize=None)
def _make_gather(b_total: int, d: int):
    assert b_total % (_NW * _CHUNK) == 0
    b_per_w = b_total // _NW
    n_chunks = b_per_w // _CHUNK
    mesh = plsc.VectorSubcoreMesh(core_axis_name="c", subcore_axis_name="s")

    @functools.partial(
        pl.kernel,
        out_type=jax.ShapeDtypeStruct((b_total, d), jnp.float32),
        mesh=mesh,
        scratch_types=[
            pltpu.VMEM((n_chunks, _CHUNK), jnp.int32),
            pltpu.VMEM((_NBUF, _CHUNK, d), jnp.float32),
            pltpu.SemaphoreType.DMA((_NBUF,)),
            pltpu.SemaphoreType.DMA((_NBUF,)),
        ],
        compiler_params=pltpu.CompilerParams(use_tc_tiling_on_sc=False),
    )
    def gather_kernel(idx_hbm, table_hbm, out_hbm, idx_v, rows_v, gsem, ssem):
        wid = lax.axis_index("s") * _NUM_CORES + lax.axis_index("c")
        base = wid * b_per_w
        pltpu.sync_copy(idx_hbm.at[wid], idx_v)

        def gather_chunk(i, b):
            return pltpu.make_async_copy(
                table_hbm.at[idx_v.at[i]], rows_v.at[b], gsem.at[b])

        def store_chunk(i, b):
            return pltpu.make_async_copy(
                rows_v.at[b], out_hbm.at[pl.ds(base + i * _CHUNK, _CHUNK)],
                ssem.at[b])

        gather_chunk(0, 0).start()
        gather_chunk(1, 1).start()

        def body(j, carry):
            for b in range(_NBUF):
                i = j * _NBUF + b
                b2 = (b + 2) % _NBUF
                gather_chunk(i, b).wait()
                store_chunk(i, b).start()
                pl.when(i >= 2)(lambda: store_chunk(i - 2, b2).wait())
                pl.when(i + 2 < n_chunks)(lambda: gather_chunk(i + 2, b2).start())
            return carry

        lax.fori_loop(0, n_chunks // _NBUF, body, 0)
        store_chunk(n_chunks - 2, (n_chunks - 2) % _NBUF).wait()
        store_chunk(n_chunks - 1, (n_chunks - 1) % _NBUF).wait()

    return gather_kernel


def kernel(token_ids, weight):
    b, t = token_ids.shape
    d = weight.shape[1]
    idx = token_ids.astype(jnp.int32).reshape(_NW, -1, _CHUNK)
    out = _make_gather(b * t, d)(idx, weight)
    return out.reshape(b, t, d)


# speedup vs baseline: 1.8749x; 1.0065x over previous
import functools

import jax
import jax.numpy as jnp
from jax import lax
from jax.experimental import pallas as pl
from jax.experimental.pallas import tpu as pltpu
from jax.experimental.pallas import tpu_sc as plsc

_NUM_CORES = 2
_NUM_SUBCORES = 16
_NW = _NUM_CORES * _NUM_SUBCORES
_CHUNK = 128
_NBUF = 8
_LOOKAHEAD = 6


@functools.lru_cache(maxsize=None)
def _make_gather(b_total: int, d: int):
    assert b_total % (_NW * _CHUNK) == 0
    b_per_w = b_total // _NW
    n_chunks = b_per_w // _CHUNK
    mesh = plsc.VectorSubcoreMesh(core_axis_name="c", subcore_axis_name="s")

    @functools.partial(
        pl.kernel,
        out_type=jax.ShapeDtypeStruct((b_total, d), jnp.float32),
        mesh=mesh,
        scratch_types=[
            pltpu.VMEM((n_chunks, _CHUNK), jnp.int32),
            pltpu.VMEM((_NBUF, _CHUNK, d), jnp.float32),
            pltpu.SemaphoreType.DMA((_NBUF,)),
            pltpu.SemaphoreType.DMA((_NBUF,)),
        ],
        compiler_params=pltpu.CompilerParams(use_tc_tiling_on_sc=False),
    )
    def gather_kernel(idx_hbm, table_hbm, out_hbm, idx_v, rows_v, gsem, ssem):
        wid = lax.axis_index("s") * _NUM_CORES + lax.axis_index("c")
        base = wid * b_per_w
        pltpu.sync_copy(idx_hbm.at[wid], idx_v)

        def gather_chunk(i, b):
            return pltpu.make_async_copy(
                table_hbm.at[idx_v.at[i]], rows_v.at[b], gsem.at[b])

        def store_chunk(i, b):
            return pltpu.make_async_copy(
                rows_v.at[b], out_hbm.at[pl.ds(base + i * _CHUNK, _CHUNK)],
                ssem.at[b])

        for i0 in range(_LOOKAHEAD):
            gather_chunk(i0, i0 % _NBUF).start()

        def body(j, carry):
            for b in range(_NBUF):
                i = j * _NBUF + b
                b2 = (b + _LOOKAHEAD) % _NBUF
                gather_chunk(i, b).wait()
                store_chunk(i, b).start()
                pl.when(i >= _NBUF - _LOOKAHEAD)(
                    lambda: store_chunk(i + _LOOKAHEAD - _NBUF, b2).wait())
                pl.when(i + _LOOKAHEAD < n_chunks)(
                    lambda: gather_chunk(i + _LOOKAHEAD, b2).start())
            return carry

        lax.fori_loop(0, n_chunks // _NBUF, body, 0)
        for i0 in range(n_chunks - (_NBUF - _LOOKAHEAD), n_chunks):
            store_chunk(i0, i0 % _NBUF).wait()

    return gather_kernel


def kernel(token_ids, weight):
    b, t = token_ids.shape
    d = weight.shape[1]
    idx = token_ids.astype(jnp.int32).reshape(_NW, -1, _CHUNK)
    out = _make_gather(b * t, d)(idx, weight)
    return out.reshape(b, t, d)
